# baseline (speedup 1.0000x reference)
import jax
import jax.numpy as jnp
from jax.experimental import pallas as pl
from jax.experimental.pallas import tpu as pltpu

_RATE = 0.8
_ALPHA = 0.8
_BETA = 1.2


def _body(x_ref, o_ref, acc_ref, *, k, inv_n):
    j = pl.program_id(1)
    nj = pl.num_programs(1)

    @pl.when(j == 0)
    def _():
        acc_ref[...] = jnp.zeros_like(acc_ref)

    acc_ref[...] += jnp.sum(x_ref[...], axis=2)

    @pl.when(j == nj - 1)
    def _():
        fv = acc_ref[...] * inv_n
        gt = (fv[:, None, :] > fv[:, :, None]).astype(jnp.float32)
        cnt = jnp.sum(gt, axis=2)
        o_ref[0, ...] = jnp.where(cnt < k, _ALPHA, _BETA).astype(jnp.float32)


def kernel(inputs):
    b, c, h, w = inputs.shape
    n = h * w
    x = inputs.reshape(b, c, n)
    k = int(c * _RATE)

    bb = b // 2
    chunk = 1792
    steps = n // chunk

    import functools
    body = functools.partial(_body, k=k, inv_n=1.0 / n)

    out = pl.pallas_call(
        body,
        grid=(b // bb, steps),
        in_specs=[
            pl.BlockSpec((bb, c, chunk), lambda i, j: (i, 0, j)),
        ],
        out_specs=pl.BlockSpec((1, bb, c), lambda i, j: (i, 0, 0)),
        out_shape=jax.ShapeDtypeStruct((b // bb, bb, c), jnp.float32),
        scratch_shapes=[pltpu.VMEM((bb, c), jnp.float32)],
        compiler_params=pltpu.CompilerParams(
            dimension_semantics=("parallel", "arbitrary"),
        ),
    )(x)
    return out.reshape(b, c)
